# baseline (speedup 1.0000x reference)
import functools

import jax
import jax.numpy as jnp
from jax import lax
from jax.experimental import pallas as pl
from jax.experimental.pallas import tpu as pltpu
from jax.experimental.pallas import tpu_sc as plsc

_N = 4096
_D = 768
_E = 8
_H = 512
_TOP_K = 2
_DIVERSITY_PENALTY = 0.01

_T = 1024

_B = 2 * _N
_NW = 32
_BPW = _B // _NW
_CH = 64


def _sc_gather_rows(x, idx):
    mesh = plsc.VectorSubcoreMesh(core_axis_name="c", subcore_axis_name="s")

    @functools.partial(
        pl.kernel,
        out_type=jax.ShapeDtypeStruct((_B, _D), jnp.float32),
        mesh=mesh,
        scratch_types=[
            pltpu.VMEM((_CH,), jnp.int32),
            pltpu.VMEM((_CH, _D), jnp.float32),
            pltpu.SemaphoreType.DMA,
        ],
    )
    def gather_kernel(x_hbm, idx_hbm, out_hbm, idx_v, rows_v, sem):
        wid = lax.axis_index("s") * 2 + lax.axis_index("c")
        base = wid * _BPW
        for c in range(_BPW // _CH):
            off = base + c * _CH
            pltpu.sync_copy(idx_hbm.at[pl.ds(off, _CH)], idx_v)
            pltpu.async_copy(x_hbm.at[idx_v], rows_v, sem).wait()
            pltpu.sync_copy(rows_v, out_hbm.at[pl.ds(off, _CH)])

    return gather_kernel(x, idx)


def _moe_block_kernel(x_ref, gw_ref, gb_ref, w1_ref, b1_ref, w2_ref, b2_ref,
                      out_ref, usage_ref):
    x = x_ref[...]
    logits = jnp.dot(x, gw_ref[...], preferred_element_type=jnp.float32)
    logits = logits + gb_ref[...]
    s = jax.nn.softmax(logits, axis=-1)
    usage_ref[0, :, :] = jnp.sum(s, axis=0, keepdims=True)

    eids = jax.lax.broadcasted_iota(jnp.int32, s.shape, 1)
    i1 = jnp.argmax(s, axis=-1)
    s1 = jnp.max(s, axis=-1)
    s_masked = jnp.where(eids == i1[:, None], -jnp.inf, s)
    i2 = jnp.argmax(s_masked, axis=-1)
    s2 = jnp.max(s_masked, axis=-1)
    denom = s1 + s2
    combine = (jnp.where(eids == i1[:, None], (s1 / denom)[:, None], 0.0)
               + jnp.where(eids == i2[:, None], (s2 / denom)[:, None], 0.0))

    acc = jnp.zeros((x.shape[0], _D), jnp.float32)
    for e in range(_E):
        h = jnp.dot(x, w1_ref[e], preferred_element_type=jnp.float32)
        h = jnp.maximum(h + b1_ref[e][None, :], 0.0)
        y = jnp.dot(h, w2_ref[e], preferred_element_type=jnp.float32)
        y = y + b2_ref[e][None, :]
        acc = acc + combine[:, e][:, None] * y
    out_ref[...] = acc


@functools.partial(jax.jit, static_argnames=())
def kernel(x, gate_W, gate_b, W1, b1, W2, b2):
    nb = _N // _T
    out, usage = pl.pallas_call(
        _moe_block_kernel,
        grid=(nb,),
        in_specs=[
            pl.BlockSpec((_T, _D), lambda i: (i, 0)),
            pl.BlockSpec((_D, _E), lambda i: (0, 0)),
            pl.BlockSpec((1, _E), lambda i: (0, 0)),
            pl.BlockSpec((_E, _D, _H), lambda i: (0, 0, 0)),
            pl.BlockSpec((_E, _H), lambda i: (0, 0)),
            pl.BlockSpec((_E, _H, _D), lambda i: (0, 0, 0)),
            pl.BlockSpec((_E, _D), lambda i: (0, 0)),
        ],
        out_specs=[
            pl.BlockSpec((_T, _D), lambda i: (i, 0)),
            pl.BlockSpec((1, 1, _E), lambda i: (i, 0, 0)),
        ],
        out_shape=[
            jax.ShapeDtypeStruct((_N, _D), jnp.float32),
            jax.ShapeDtypeStruct((nb, 1, _E), jnp.float32),
        ],
    )(x, gate_W, gate_b.reshape(1, _E), W1, b1, W2, b2)
    expert_usage = jnp.sum(usage, axis=(0, 1)) / _N
    load_balance_loss = _DIVERSITY_PENALTY * jnp.sum(expert_usage ** 2)

    probe_idx = (jnp.arange(_B, dtype=jnp.int32) * 1000003) % _N
    xs = _sc_gather_rows(x, probe_idx)
    load_balance_loss = load_balance_loss + 0.0 * xs[0, 0]
    return (out, load_balance_loss)


# speedup vs baseline: 1.3589x; 1.3589x over previous
import functools

import jax
import jax.numpy as jnp
from jax.experimental import pallas as pl
from jax.experimental.pallas import tpu as pltpu

_N = 4096
_D = 768
_E = 8
_H = 512
_TOP_K = 2
_DIVERSITY_PENALTY = 0.01

_T = 1024


def _moe_block_kernel(x_ref, gw_ref, gb_ref, w1_ref, b1_ref, w2_ref, b2_ref,
                      out_ref, loss_ref, usage_scr):
    i = pl.program_id(0)
    x = x_ref[...]
    logits = jnp.dot(x, gw_ref[...], preferred_element_type=jnp.float32)
    logits = logits + gb_ref[...]
    s = jax.nn.softmax(logits, axis=-1)

    usum = jnp.sum(s, axis=0, keepdims=True)

    @pl.when(i == 0)
    def _init_usage():
        usage_scr[...] = usum

    @pl.when(i > 0)
    def _acc_usage():
        usage_scr[...] += usum

    @pl.when(i == pl.num_programs(0) - 1)
    def _loss():
        u = usage_scr[...] / _N
        loss_ref[0, 0] = _DIVERSITY_PENALTY * jnp.sum(u * u)

    eids = jax.lax.broadcasted_iota(jnp.int32, s.shape, 1)
    i1 = jnp.argmax(s, axis=-1)
    s1 = jnp.max(s, axis=-1)
    s_masked = jnp.where(eids == i1[:, None], -jnp.inf, s)
    i2 = jnp.argmax(s_masked, axis=-1)
    s2 = jnp.max(s_masked, axis=-1)
    denom = s1 + s2
    combine = (jnp.where(eids == i1[:, None], (s1 / denom)[:, None], 0.0)
               + jnp.where(eids == i2[:, None], (s2 / denom)[:, None], 0.0))

    acc = jnp.zeros((x.shape[0], _D), jnp.float32)
    for e in range(_E):
        h = jnp.dot(x, w1_ref[e], preferred_element_type=jnp.float32)
        h = jnp.maximum(h + b1_ref[e][None, :], 0.0)
        y = jnp.dot(h, w2_ref[e], preferred_element_type=jnp.float32)
        y = y + b2_ref[e][None, :]
        acc = acc + combine[:, e][:, None] * y
    out_ref[...] = acc


@functools.partial(jax.jit, static_argnames=())
def kernel(x, gate_W, gate_b, W1, b1, W2, b2):
    nb = _N // _T
    out, loss = pl.pallas_call(
        _moe_block_kernel,
        grid=(nb,),
        in_specs=[
            pl.BlockSpec((_T, _D), lambda i: (i, 0)),
            pl.BlockSpec((_D, _E), lambda i: (0, 0)),
            pl.BlockSpec((1, _E), lambda i: (0, 0)),
            pl.BlockSpec((_E, _D, _H), lambda i: (0, 0, 0)),
            pl.BlockSpec((_E, _H), lambda i: (0, 0)),
            pl.BlockSpec((_E, _H, _D), lambda i: (0, 0, 0)),
            pl.BlockSpec((_E, _D), lambda i: (0, 0)),
        ],
        out_specs=[
            pl.BlockSpec((_T, _D), lambda i: (i, 0)),
            pl.BlockSpec((1, 1), lambda i: (0, 0),
                         memory_space=pltpu.MemorySpace.SMEM),
        ],
        out_shape=[
            jax.ShapeDtypeStruct((_N, _D), jnp.float32),
            jax.ShapeDtypeStruct((1, 1), jnp.float32),
        ],
        scratch_shapes=[
            pltpu.VMEM((1, _E), jnp.float32),
        ],
    )(x, gate_W, gate_b.reshape(1, _E), W1, b1, W2, b2)
    return (out, loss[0, 0])


# speedup vs baseline: 1.3731x; 1.0105x over previous
import functools

import jax
import jax.numpy as jnp
from jax.experimental import pallas as pl
from jax.experimental.pallas import tpu as pltpu

_N = 4096
_D = 768
_E = 8
_H = 512
_TOP_K = 2
_DIVERSITY_PENALTY = 0.01

_T = 1024


def _moe_block_kernel(x_ref, gw_ref, gb_ref, w1_ref, b1_ref, w2_ref, b2_ref,
                      out_ref, loss_ref, usage_scr):
    i = pl.program_id(0)
    x = x_ref[...]
    logits = jnp.dot(x, gw_ref[...], preferred_element_type=jnp.float32)
    logits = logits + gb_ref[...]
    s = jax.nn.softmax(logits, axis=-1)

    usum = jnp.sum(s, axis=0, keepdims=True)

    @pl.when(i == 0)
    def _init_usage():
        usage_scr[...] = usum

    @pl.when(i > 0)
    def _acc_usage():
        usage_scr[...] += usum

    @pl.when(i == pl.num_programs(0) - 1)
    def _loss():
        u = usage_scr[...] / _N
        loss_ref[0, 0] = _DIVERSITY_PENALTY * jnp.sum(u * u)

    eids = jax.lax.broadcasted_iota(jnp.int32, s.shape, 1)
    i1 = jnp.argmax(s, axis=-1)
    s1 = jnp.max(s, axis=-1)
    s_masked = jnp.where(eids == i1[:, None], -jnp.inf, s)
    i2 = jnp.argmax(s_masked, axis=-1)
    s2 = jnp.max(s_masked, axis=-1)
    denom = s1 + s2
    combine = (jnp.where(eids == i1[:, None], (s1 / denom)[:, None], 0.0)
               + jnp.where(eids == i2[:, None], (s2 / denom)[:, None], 0.0))

    acc = jnp.zeros((x.shape[0], _D), jnp.float32)
    for e0 in range(0, _E, 2):
        hs = []
        for e in (e0, e0 + 1):
            h = jnp.dot(x, w1_ref[e], preferred_element_type=jnp.float32)
            hs.append(jnp.maximum(h + b1_ref[e][None, :], 0.0))
        for k, e in enumerate((e0, e0 + 1)):
            y = jnp.dot(hs[k], w2_ref[e], preferred_element_type=jnp.float32)
            y = y + b2_ref[e][None, :]
            acc = acc + combine[:, e][:, None] * y
    out_ref[...] = acc


@functools.partial(jax.jit, static_argnames=())
def kernel(x, gate_W, gate_b, W1, b1, W2, b2):
    nb = _N // _T
    out, loss = pl.pallas_call(
        _moe_block_kernel,
        grid=(nb,),
        in_specs=[
            pl.BlockSpec((_T, _D), lambda i: (i, 0)),
            pl.BlockSpec((_D, _E), lambda i: (0, 0)),
            pl.BlockSpec((1, _E), lambda i: (0, 0)),
            pl.BlockSpec((_E, _D, _H), lambda i: (0, 0, 0)),
            pl.BlockSpec((_E, _H), lambda i: (0, 0)),
            pl.BlockSpec((_E, _H, _D), lambda i: (0, 0, 0)),
            pl.BlockSpec((_E, _D), lambda i: (0, 0)),
        ],
        out_specs=[
            pl.BlockSpec((_T, _D), lambda i: (i, 0)),
            pl.BlockSpec((1, 1), lambda i: (0, 0),
                         memory_space=pltpu.MemorySpace.SMEM),
        ],
        out_shape=[
            jax.ShapeDtypeStruct((_N, _D), jnp.float32),
            jax.ShapeDtypeStruct((1, 1), jnp.float32),
        ],
        scratch_shapes=[
            pltpu.VMEM((1, _E), jnp.float32),
        ],
    )(x, gate_W, gate_b.reshape(1, _E), W1, b1, W2, b2)
    return (out, loss[0, 0])
